# Initial kernel scaffold; baseline (speedup 1.0000x reference)
import jax
import jax.numpy as jnp
from jax.experimental import pallas as pl


def kernel(observations):
    raise NotImplementedError("write your pallas kernel here")


# speedup vs baseline: 7.7316x; 7.7316x over previous
import jax
import jax.numpy as jnp
import numpy as np
from jax.experimental import pallas as pl
from jax.experimental.pallas import tpu as pltpu

N_PART = 1024
BB = 256


def _filter_body(coef_ref, obs_ref, p0_ref, noise_ref, est_ref, ess_ref,
                 part_s, w_s):
    t = pl.program_id(1)

    @pl.when(t == 0)
    def _init():
        part_s[...] = p0_ref[...]
        w_s[...] = jnp.full((BB, N_PART), 1.0 / N_PART, jnp.float32)

    p = part_s[...]
    n = noise_ref[0]
    c = coef_ref[0, t]
    term1 = 0.5 * p
    term2 = 25.0 * p / (1.0 + p * p)
    p = term1 + term2 + c + n
    part_s[...] = p

    T = obs_ref.shape[1]
    tmask = jax.lax.broadcasted_iota(jnp.int32, (BB, T), 1) == t
    obs_col = jnp.sum(jnp.where(tmask, obs_ref[...], 0.0), axis=1)
    pred = p * p / 20.0
    d = pred - obs_col[:, None]
    lik = jnp.exp(-0.5 * (d * d)) + 1e-10
    w = w_s[...] * lik
    denom = jnp.sum(w, axis=1, keepdims=True) + 1e-10
    w = w / denom
    w_s[...] = w
    est_ref[0, 0, :] = jnp.sum(w * p, axis=1)
    ess_ref[0, 0, :] = 1.0 / jnp.sum(w * w, axis=1)


def kernel(observations):
    B, T, D = observations.shape
    key = jax.random.key(42)
    k0, kloop = jax.random.split(key)
    particles0 = jax.random.normal(k0, (B, N_PART, D), dtype=jnp.float32) * np.sqrt(5.0)
    keys = jax.random.split(kloop, T)
    noise_all = jax.vmap(
        lambda k: jax.random.normal(k, (B, N_PART, D), dtype=jnp.float32))(keys)
    ts = jnp.arange(T, dtype=jnp.float32)
    coef = (8.0 * jnp.cos(1.2 * ts)).reshape(1, T)

    p0 = particles0[:, :, 0]
    noise2 = noise_all[:, :, :, 0]
    obs2 = observations[:, :, 0]

    nb = B // BB
    est, ess = pl.pallas_call(
        _filter_body,
        grid=(nb, T),
        in_specs=[
            pl.BlockSpec(memory_space=pltpu.SMEM),
            pl.BlockSpec((BB, T), lambda ib, it: (ib, 0)),
            pl.BlockSpec((BB, N_PART), lambda ib, it: (ib, 0)),
            pl.BlockSpec((1, BB, N_PART), lambda ib, it: (it, ib, 0)),
        ],
        out_specs=[
            pl.BlockSpec((1, 1, BB), lambda ib, it: (it, 0, ib)),
            pl.BlockSpec((1, 1, BB), lambda ib, it: (it, 0, ib)),
        ],
        out_shape=[
            jax.ShapeDtypeStruct((T, 1, B), jnp.float32),
            jax.ShapeDtypeStruct((T, 1, B), jnp.float32),
        ],
        scratch_shapes=[
            pltpu.VMEM((BB, N_PART), jnp.float32),
            pltpu.VMEM((BB, N_PART), jnp.float32),
        ],
    )(coef, obs2, p0, noise2)

    est_states = est[:, 0, :].T[:, :, None]
    ess_history = ess[:, 0, :].T
    return (est_states, ess_history)


# speedup vs baseline: 7.8197x; 1.0114x over previous
import jax
import jax.numpy as jnp
import numpy as np
from jax.experimental import pallas as pl
from jax.experimental.pallas import tpu as pltpu

N_PART = 1024
BB = 256

_LO = np.float32(np.nextafter(np.float32(-1.0), np.float32(0.0)))
_SCALE = np.float32(np.float32(1.0) - _LO)
_SQRT2 = np.float32(np.sqrt(2.0))
_SQRT10 = np.float32(np.float32(np.sqrt(2.0)) * np.float32(np.sqrt(5.0)))


def _rotl(x, r):
    return (x << jnp.uint32(r)) | (x >> jnp.uint32(32 - r))


def _threefry_rounds(x0, x1, rots):
    for r in rots:
        x0 = x0 + x1
        x1 = _rotl(x1, r)
        x1 = x0 ^ x1
    return x0, x1


def _normal_block(k1, k2, base, scale=_SQRT2):
    row = jax.lax.broadcasted_iota(jnp.int32, (BB, N_PART), 0)
    col = jax.lax.broadcasted_iota(jnp.int32, (BB, N_PART), 1)
    idx = base + row * N_PART + col
    x1 = jax.lax.bitcast_convert_type(idx, jnp.uint32)
    x0 = jnp.zeros((BB, N_PART), jnp.uint32)
    ks0, ks1 = k1, k2
    ks2 = k1 ^ k2 ^ jnp.uint32(0x1BD11BDA)
    R0 = (13, 15, 26, 6)
    R1 = (17, 29, 16, 24)
    x0 = x0 + ks0
    x1 = x1 + ks1
    x0, x1 = _threefry_rounds(x0, x1, R0)
    x0 = x0 + ks1
    x1 = x1 + ks2 + jnp.uint32(1)
    x0, x1 = _threefry_rounds(x0, x1, R1)
    x0 = x0 + ks2
    x1 = x1 + ks0 + jnp.uint32(2)
    x0, x1 = _threefry_rounds(x0, x1, R0)
    x0 = x0 + ks0
    x1 = x1 + ks1 + jnp.uint32(3)
    x0, x1 = _threefry_rounds(x0, x1, R1)
    x0 = x0 + ks1
    x1 = x1 + ks2 + jnp.uint32(4)
    x0, x1 = _threefry_rounds(x0, x1, R0)
    x0 = x0 + ks2
    x1 = x1 + ks0 + jnp.uint32(5)
    bits = x0 ^ x1
    fb = (bits >> jnp.uint32(9)) | jnp.uint32(0x3F800000)
    f = jax.lax.bitcast_convert_type(fb, jnp.float32) - np.float32(1.0)
    u = jnp.maximum(_LO, f * _SCALE + _LO)
    return scale * jax.lax.erf_inv(u)


def _filter_body(keys_ref, coef_ref, obs_ref, est_ref, ess_ref, part_s, w_s):
    ib = pl.program_id(0)
    t = pl.program_id(1)
    base = (ib * BB) * N_PART

    @pl.when(t == 0)
    def _init():
        part_s[...] = _normal_block(keys_ref[0, 0], keys_ref[0, 1], base,
                                    scale=_SQRT10)
        w_s[...] = jnp.full((BB, N_PART), 1.0 / N_PART, jnp.float32)

    n = _normal_block(keys_ref[t + 1, 0], keys_ref[t + 1, 1], base)
    p = part_s[...]
    c = coef_ref[0, t]
    term1 = 0.5 * p
    term2 = 25.0 * p / (1.0 + p * p)
    p = term1 + term2 + c + n
    part_s[...] = p

    T = obs_ref.shape[1]
    tmask = jax.lax.broadcasted_iota(jnp.int32, (BB, T), 1) == t
    obs_col = jnp.sum(jnp.where(tmask, obs_ref[...], 0.0), axis=1)
    pred = p * p / 20.0
    d = pred - obs_col[:, None]
    lik = jnp.exp(-0.5 * (d * d)) + 1e-10
    w = w_s[...] * lik
    denom = jnp.sum(w, axis=1, keepdims=True) + 1e-10
    w = w / denom
    w_s[...] = w
    est_ref[0, 0, :] = jnp.sum(w * p, axis=1)
    ess_ref[0, 0, :] = 1.0 / jnp.sum(w * w, axis=1)


def kernel(observations):
    B, T, D = observations.shape
    key = jax.random.key(42)
    k0, kloop = jax.random.split(key)
    keys = jax.random.split(kloop, T)
    keydata = jnp.concatenate(
        [jax.random.key_data(k0)[None], jax.random.key_data(keys)], axis=0
    ).astype(jnp.uint32)
    ts = jnp.arange(T, dtype=jnp.float32)
    coef = (8.0 * jnp.cos(1.2 * ts)).reshape(1, T)
    obs2 = observations[:, :, 0]

    nb = B // BB
    est, ess = pl.pallas_call(
        _filter_body,
        grid=(nb, T),
        in_specs=[
            pl.BlockSpec(memory_space=pltpu.SMEM),
            pl.BlockSpec(memory_space=pltpu.SMEM),
            pl.BlockSpec((BB, T), lambda ib, it: (ib, 0)),
        ],
        out_specs=[
            pl.BlockSpec((1, 1, BB), lambda ib, it: (it, 0, ib)),
            pl.BlockSpec((1, 1, BB), lambda ib, it: (it, 0, ib)),
        ],
        out_shape=[
            jax.ShapeDtypeStruct((T, 1, B), jnp.float32),
            jax.ShapeDtypeStruct((T, 1, B), jnp.float32),
        ],
        scratch_shapes=[
            pltpu.VMEM((BB, N_PART), jnp.float32),
            pltpu.VMEM((BB, N_PART), jnp.float32),
        ],
    )(keydata, coef, obs2)

    est_states = est[:, 0, :].T[:, :, None]
    ess_history = ess[:, 0, :].T
    return (est_states, ess_history)


# speedup vs baseline: 7.9126x; 1.0119x over previous
import jax
import jax.numpy as jnp
import numpy as np
from jax.experimental import pallas as pl
from jax.experimental.pallas import tpu as pltpu

N_PART = 1024
BB = 512

_LO = np.float32(np.nextafter(np.float32(-1.0), np.float32(0.0)))
_SCALE = np.float32(np.float32(1.0) - _LO)
_SQRT2 = np.float32(np.sqrt(2.0))
_SQRT10 = np.float32(np.float32(np.sqrt(2.0)) * np.float32(np.sqrt(5.0)))


def _rotl(x, r):
    return (x << jnp.uint32(r)) | (x >> jnp.uint32(32 - r))


def _threefry_rounds(x0, x1, rots):
    for r in rots:
        x0 = x0 + x1
        x1 = _rotl(x1, r)
        x1 = x0 ^ x1
    return x0, x1


def _normal_block(k1, k2, base, scale=_SQRT2):
    row = jax.lax.broadcasted_iota(jnp.int32, (BB, N_PART), 0)
    col = jax.lax.broadcasted_iota(jnp.int32, (BB, N_PART), 1)
    idx = base + row * N_PART + col
    x1 = jax.lax.bitcast_convert_type(idx, jnp.uint32)
    x0 = jnp.zeros((BB, N_PART), jnp.uint32)
    ks0, ks1 = k1, k2
    ks2 = k1 ^ k2 ^ jnp.uint32(0x1BD11BDA)
    R0 = (13, 15, 26, 6)
    R1 = (17, 29, 16, 24)
    x0 = x0 + ks0
    x1 = x1 + ks1
    x0, x1 = _threefry_rounds(x0, x1, R0)
    x0 = x0 + ks1
    x1 = x1 + ks2 + jnp.uint32(1)
    x0, x1 = _threefry_rounds(x0, x1, R1)
    x0 = x0 + ks2
    x1 = x1 + ks0 + jnp.uint32(2)
    x0, x1 = _threefry_rounds(x0, x1, R0)
    x0 = x0 + ks0
    x1 = x1 + ks1 + jnp.uint32(3)
    x0, x1 = _threefry_rounds(x0, x1, R1)
    x0 = x0 + ks1
    x1 = x1 + ks2 + jnp.uint32(4)
    x0, x1 = _threefry_rounds(x0, x1, R0)
    x0 = x0 + ks2
    x1 = x1 + ks0 + jnp.uint32(5)
    bits = x0 ^ x1
    fb = (bits >> jnp.uint32(9)) | jnp.uint32(0x3F800000)
    f = jax.lax.bitcast_convert_type(fb, jnp.float32) - np.float32(1.0)
    u = jnp.maximum(_LO, f * _SCALE + _LO)
    return scale * jax.lax.erf_inv(u)


def _filter_body(keys_ref, coef_ref, obs_ref, est_ref, ess_ref, part_s, w_s):
    ib = pl.program_id(0)
    t = pl.program_id(1)
    base = (ib * BB) * N_PART

    @pl.when(t == 0)
    def _init():
        part_s[...] = _normal_block(keys_ref[0, 0], keys_ref[0, 1], base,
                                    scale=_SQRT10)
        w_s[...] = jnp.full((BB, N_PART), 1.0 / N_PART, jnp.float32)

    n = _normal_block(keys_ref[t + 1, 0], keys_ref[t + 1, 1], base)
    p = part_s[...]
    c = coef_ref[0, t]
    term1 = 0.5 * p
    term2 = 25.0 * p / (1.0 + p * p)
    p = term1 + term2 + c + n
    part_s[...] = p

    T = obs_ref.shape[1]
    tmask = jax.lax.broadcasted_iota(jnp.int32, (BB, T), 1) == t
    obs_col = jnp.sum(jnp.where(tmask, obs_ref[...], 0.0), axis=1)
    pred = (p * p) * 0.05
    d = pred - obs_col[:, None]
    lik = jnp.exp(-0.5 * (d * d)) + 1e-10
    w = w_s[...] * lik
    denom = jnp.sum(w, axis=1, keepdims=True) + 1e-10
    w = w * (1.0 / denom)
    w_s[...] = w
    est_ref[0, 0, :] = jnp.sum(w * p, axis=1)
    ess_ref[0, 0, :] = 1.0 / jnp.sum(w * w, axis=1)


def kernel(observations):
    B, T, D = observations.shape
    key = jax.random.key(42)
    k0, kloop = jax.random.split(key)
    keys = jax.random.split(kloop, T)
    keydata = jnp.concatenate(
        [jax.random.key_data(k0)[None], jax.random.key_data(keys)], axis=0
    ).astype(jnp.uint32)
    ts = jnp.arange(T, dtype=jnp.float32)
    coef = (8.0 * jnp.cos(1.2 * ts)).reshape(1, T)
    obs2 = observations[:, :, 0]

    nb = B // BB
    est, ess = pl.pallas_call(
        _filter_body,
        grid=(nb, T),
        in_specs=[
            pl.BlockSpec(memory_space=pltpu.SMEM),
            pl.BlockSpec(memory_space=pltpu.SMEM),
            pl.BlockSpec((BB, T), lambda ib, it: (ib, 0)),
        ],
        out_specs=[
            pl.BlockSpec((1, 1, BB), lambda ib, it: (it, 0, ib)),
            pl.BlockSpec((1, 1, BB), lambda ib, it: (it, 0, ib)),
        ],
        out_shape=[
            jax.ShapeDtypeStruct((T, 1, B), jnp.float32),
            jax.ShapeDtypeStruct((T, 1, B), jnp.float32),
        ],
        scratch_shapes=[
            pltpu.VMEM((BB, N_PART), jnp.float32),
            pltpu.VMEM((BB, N_PART), jnp.float32),
        ],
    )(keydata, coef, obs2)

    est_states = est[:, 0, :].T[:, :, None]
    ess_history = ess[:, 0, :].T
    return (est_states, ess_history)


# speedup vs baseline: 8.1403x; 1.0288x over previous
import jax
import jax.numpy as jnp
import numpy as np
from jax.experimental import pallas as pl
from jax.experimental.pallas import tpu as pltpu

N_PART = 1024
BB = 512

_LO = np.float32(np.nextafter(np.float32(-1.0), np.float32(0.0)))
_SCALE = np.float32(np.float32(1.0) - _LO)
_SQRT2 = np.float32(np.sqrt(2.0))
_SQRT10 = np.float32(np.float32(np.sqrt(2.0)) * np.float32(np.sqrt(5.0)))


def _rotl(x, r):
    return (x << jnp.uint32(r)) | (x >> jnp.uint32(32 - r))


def _threefry_rounds(x0, x1, rots):
    for r in rots:
        x0 = x0 + x1
        x1 = _rotl(x1, r)
        x1 = x0 ^ x1
    return x0, x1


def _normal_block(k1, k2, base, scale=_SQRT2):
    row = jax.lax.broadcasted_iota(jnp.int32, (BB, N_PART), 0)
    col = jax.lax.broadcasted_iota(jnp.int32, (BB, N_PART), 1)
    idx = base + row * N_PART + col
    x1 = jax.lax.bitcast_convert_type(idx, jnp.uint32)
    x0 = jnp.zeros((BB, N_PART), jnp.uint32)
    ks0, ks1 = k1, k2
    ks2 = k1 ^ k2 ^ jnp.uint32(0x1BD11BDA)
    R0 = (13, 15, 26, 6)
    R1 = (17, 29, 16, 24)
    x0 = x0 + ks0
    x1 = x1 + ks1
    x0, x1 = _threefry_rounds(x0, x1, R0)
    x0 = x0 + ks1
    x1 = x1 + ks2 + jnp.uint32(1)
    x0, x1 = _threefry_rounds(x0, x1, R1)
    x0 = x0 + ks2
    x1 = x1 + ks0 + jnp.uint32(2)
    x0, x1 = _threefry_rounds(x0, x1, R0)
    x0 = x0 + ks0
    x1 = x1 + ks1 + jnp.uint32(3)
    x0, x1 = _threefry_rounds(x0, x1, R1)
    x0 = x0 + ks1
    x1 = x1 + ks2 + jnp.uint32(4)
    x0, x1 = _threefry_rounds(x0, x1, R0)
    x0 = x0 + ks2
    x1 = x1 + ks0 + jnp.uint32(5)
    bits = x0 ^ x1
    fb = (bits >> jnp.uint32(9)) | jnp.uint32(0x3F800000)
    f = jax.lax.bitcast_convert_type(fb, jnp.float32) - np.float32(1.0)
    u = f * _SCALE + _LO
    return scale * _erf_inv(u)


_W5 = [np.float32(v) for v in (
    2.81022636e-08, 3.43273939e-07, -3.5233877e-06,
    -4.39150654e-06, 0.00021858087, -0.00125372503,
    -0.00417768164, 0.246640727, 1.50140941)]
_G5 = [np.float32(v) for v in (
    -0.000200214257, 0.000100950558, 0.00134934322,
    -0.00367342844, 0.00573950773, -0.0076224613,
    0.00943887047, 1.00167406, 2.83297682)]


def _erf_inv(x):
    w = -jnp.log1p(x * -x)
    w_lt_5 = w < 5.0
    w = jnp.where(w_lt_5, w - 2.5, jnp.sqrt(w) - 3.0)
    p = jnp.where(w_lt_5, _W5[0], _G5[0])
    for i in range(1, 9):
        c = jnp.where(w_lt_5, _W5[i], _G5[i])
        p = c + p * w
    return p * x


def _filter_body(keys_ref, coef_ref, obs_ref, est_ref, ess_ref, part_s, w_s):
    ib = pl.program_id(0)
    t = pl.program_id(1)
    base = (ib * BB) * N_PART

    @pl.when(t == 0)
    def _init():
        part_s[...] = _normal_block(keys_ref[0, 0], keys_ref[0, 1], base,
                                    scale=_SQRT10)
        w_s[...] = jnp.full((BB, N_PART), 1.0 / N_PART, jnp.float32)

    n = _normal_block(keys_ref[t + 1, 0], keys_ref[t + 1, 1], base)
    p = part_s[...]
    c = coef_ref[0, t]
    term1 = 0.5 * p
    term2 = 25.0 * p / (1.0 + p * p)
    p = term1 + term2 + c + n
    part_s[...] = p

    T = obs_ref.shape[1]
    tmask = jax.lax.broadcasted_iota(jnp.int32, (BB, T), 1) == t
    obs_col = jnp.sum(jnp.where(tmask, obs_ref[...], 0.0), axis=1)
    pred = (p * p) * 0.05
    d = pred - obs_col[:, None]
    lik = jnp.exp(-0.5 * (d * d)) + 1e-10
    w = w_s[...] * lik
    denom = jnp.sum(w, axis=1, keepdims=True) + 1e-10
    w = w * (1.0 / denom)
    w_s[...] = w
    est_ref[0, 0, :] = jnp.sum(w * p, axis=1)
    ess_ref[0, 0, :] = 1.0 / jnp.sum(w * w, axis=1)


def kernel(observations):
    B, T, D = observations.shape
    key = jax.random.key(42)
    k0, kloop = jax.random.split(key)
    keys = jax.random.split(kloop, T)
    keydata = jnp.concatenate(
        [jax.random.key_data(k0)[None], jax.random.key_data(keys)], axis=0
    ).astype(jnp.uint32)
    ts = jnp.arange(T, dtype=jnp.float32)
    coef = (8.0 * jnp.cos(1.2 * ts)).reshape(1, T)
    obs2 = observations[:, :, 0]

    nb = B // BB
    est, ess = pl.pallas_call(
        _filter_body,
        grid=(nb, T),
        in_specs=[
            pl.BlockSpec(memory_space=pltpu.SMEM),
            pl.BlockSpec(memory_space=pltpu.SMEM),
            pl.BlockSpec((BB, T), lambda ib, it: (ib, 0)),
        ],
        out_specs=[
            pl.BlockSpec((1, 1, BB), lambda ib, it: (it, 0, ib)),
            pl.BlockSpec((1, 1, BB), lambda ib, it: (it, 0, ib)),
        ],
        out_shape=[
            jax.ShapeDtypeStruct((T, 1, B), jnp.float32),
            jax.ShapeDtypeStruct((T, 1, B), jnp.float32),
        ],
        scratch_shapes=[
            pltpu.VMEM((BB, N_PART), jnp.float32),
            pltpu.VMEM((BB, N_PART), jnp.float32),
        ],
    )(keydata, coef, obs2)

    est_states = est[:, 0, :].T[:, :, None]
    ess_history = ess[:, 0, :].T
    return (est_states, ess_history)
